# baseline (speedup 1.0000x reference)
import functools

import jax
import jax.numpy as jnp
from jax import lax
from jax.experimental import pallas as pl
from jax.experimental.pallas import tpu as pltpu
from jax.experimental.pallas import tpu_sc as plsc

BATCH = 16384
D = 32
NC = 2
NS = 16
NW = NC * NS
BPW = BATCH // NW

_mesh = plsc.VectorSubcoreMesh(core_axis_name="c", subcore_axis_name="s")


@functools.partial(
    pl.kernel,
    mesh=_mesh,
    compiler_params=pltpu.CompilerParams(use_tc_tiling_on_sc=False),
    out_type=(
        jax.ShapeDtypeStruct((BATCH, D), jnp.float32),
        jax.ShapeDtypeStruct((BATCH, D), jnp.float32),
        jax.ShapeDtypeStruct((BATCH, D), jnp.float32),
    ),
    scratch_types=[
        pltpu.VMEM((BPW,), jnp.int32),
        pltpu.VMEM((BPW,), jnp.int32),
        pltpu.VMEM((BPW, D), jnp.float32),
        pltpu.VMEM((BPW, D), jnp.float32),
        pltpu.VMEM((BPW, D), jnp.float32),
        pltpu.VMEM((BPW, D), jnp.float32),
        pltpu.SemaphoreType.DMA,
    ],
)
def _sc_gather(gmf_user, gmf_item, mlp_user, mlp_item, uid_hbm, iid_hbm,
               gmf_out, pmlp_out, qmlp_out,
               uid_v, iid_v, gu_v, gi_v, mu_v, mi_v, sem):
    wid = lax.axis_index("s") * NC + lax.axis_index("c")
    base = wid * BPW
    pltpu.sync_copy(uid_hbm.at[pl.ds(base, BPW)], uid_v)
    pltpu.sync_copy(iid_hbm.at[pl.ds(base, BPW)], iid_v)
    c0 = pltpu.async_copy(gmf_user.at[uid_v], gu_v, sem)
    c1 = pltpu.async_copy(gmf_item.at[iid_v], gi_v, sem)
    c2 = pltpu.async_copy(mlp_user.at[uid_v], mu_v, sem)
    c3 = pltpu.async_copy(mlp_item.at[iid_v], mi_v, sem)
    c0.wait()
    c1.wait()

    def body(i, carry):
        gu_v[i, pl.ds(0, 16)] = gu_v[i, pl.ds(0, 16)] * gi_v[i, pl.ds(0, 16)]
        gu_v[i, pl.ds(16, 16)] = gu_v[i, pl.ds(16, 16)] * gi_v[i, pl.ds(16, 16)]
        return carry

    lax.fori_loop(0, BPW, body, 0)
    pltpu.sync_copy(gu_v, gmf_out.at[pl.ds(base, BPW)])
    c2.wait()
    pltpu.sync_copy(mu_v, pmlp_out.at[pl.ds(base, BPW)])
    c3.wait()
    pltpu.sync_copy(mi_v, qmlp_out.at[pl.ds(base, BPW)])


def _leaky(x):
    return jnp.where(x >= 0, x, 0.01 * x)


def _mlp_body(gmf_ref, pmlp_ref, qmlp_ref, w0u_ref, w0i_ref, b0_ref,
              w1_ref, b1_ref, w2_ref, b2_ref, wg_ref, wh_ref, out_ref):
    h = jnp.dot(pmlp_ref[...], w0u_ref[...], preferred_element_type=jnp.float32)
    h = h + jnp.dot(qmlp_ref[...], w0i_ref[...], preferred_element_type=jnp.float32)
    h = _leaky(h + b0_ref[...])
    h = _leaky(jnp.dot(h, w1_ref[...], preferred_element_type=jnp.float32) + b1_ref[...])
    h = _leaky(jnp.dot(h, w2_ref[...], preferred_element_type=jnp.float32) + b2_ref[...])
    out = jnp.dot(gmf_ref[...], wg_ref[...], preferred_element_type=jnp.float32)
    out = out + jnp.dot(h, wh_ref[...], preferred_element_type=jnp.float32)
    out_ref[...] = out


_BLK = 2048


def _mlp_call(gmf, pmlp, qmlp, w0u, w0i, b0, w1t, b1, w2t, b2, wg, wh):
    grid = (BATCH // _BLK,)
    full = lambda shape: pl.BlockSpec(shape, lambda i: (0, 0))
    return pl.pallas_call(
        _mlp_body,
        grid=grid,
        in_specs=[
            pl.BlockSpec((_BLK, D), lambda i: (i, 0)),
            pl.BlockSpec((_BLK, D), lambda i: (i, 0)),
            pl.BlockSpec((_BLK, D), lambda i: (i, 0)),
            full((D, 128)),
            full((D, 128)),
            full((1, 128)),
            full((128, 64)),
            full((1, 64)),
            full((64, 32)),
            full((1, 32)),
            full((D, 1)),
            full((32, 1)),
        ],
        out_specs=pl.BlockSpec((_BLK, 1), lambda i: (i, 0)),
        out_shape=jax.ShapeDtypeStruct((BATCH, 1), jnp.float32),
    )(gmf, pmlp, qmlp, w0u, w0i, b0, w1t, b1, w2t, b2, wg, wh)


def kernel(user_id, item_id, gmf_user, gmf_item, mlp_user, mlp_item,
           W0, b0, W1, b1, W2, b2, Wout):
    gmf, pmlp, qmlp = _sc_gather(gmf_user, gmf_item, mlp_user, mlp_item,
                                 user_id.astype(jnp.int32),
                                 item_id.astype(jnp.int32))
    w0u = W0[:, :D].T
    w0i = W0[:, D:].T
    w1t = W1.T
    w2t = W2.T
    wg = Wout[:, :D].T
    wh = Wout[:, D:].T
    return _mlp_call(gmf, pmlp, qmlp, w0u, w0i, b0.reshape(1, -1),
                     w1t, b1.reshape(1, -1), w2t, b2.reshape(1, -1), wg, wh)


# speedup vs baseline: 1.0079x; 1.0079x over previous
import functools

import jax
import jax.numpy as jnp
from jax import lax
from jax.experimental import pallas as pl
from jax.experimental.pallas import tpu as pltpu
from jax.experimental.pallas import tpu_sc as plsc

BATCH = 16384
D = 32
V = 1000001
NC = 2
NS = 16
NW = NC * NS
BPW = BATCH // NW

_PACK_BLK = 512
_PACK_GRID = (V + _PACK_BLK - 1) // _PACK_BLK

_mesh = plsc.VectorSubcoreMesh(core_axis_name="c", subcore_axis_name="s")


def _pack_body(gu_ref, gi_ref, mu_ref, mi_ref, out_ref):
    out_ref[:, 0:D] = gu_ref[...].T
    out_ref[:, D:2 * D] = gi_ref[...].T
    out_ref[:, 2 * D:3 * D] = mu_ref[...].T
    out_ref[:, 3 * D:4 * D] = mi_ref[...].T


def _pack_call(gu_t, gi_t, mu_t, mi_t):
    return pl.pallas_call(
        _pack_body,
        grid=(_PACK_GRID,),
        in_specs=[pl.BlockSpec((D, _PACK_BLK), lambda i: (0, i))] * 4,
        out_specs=pl.BlockSpec((_PACK_BLK, 4 * D), lambda i: (i, 0)),
        out_shape=jax.ShapeDtypeStruct((V, 4 * D), jnp.float32),
    )(gu_t, gi_t, mu_t, mi_t)


@functools.partial(
    pl.kernel,
    mesh=_mesh,
    compiler_params=pltpu.CompilerParams(use_tc_tiling_on_sc=True),
    out_type=(
        jax.ShapeDtypeStruct((BATCH, 4 * D), jnp.float32),
        jax.ShapeDtypeStruct((BATCH, 4 * D), jnp.float32),
    ),
    scratch_types=[
        pltpu.VMEM((BPW,), jnp.int32),
        pltpu.VMEM((BPW, 4 * D), jnp.float32),
        pltpu.SemaphoreType.DMA,
    ],
)
def _sc_gather(comb, uid_hbm, iid_hbm, u_out, i_out, idx_v, rows_v, sem):
    wid = lax.axis_index("s") * NC + lax.axis_index("c")
    base = wid * BPW
    pltpu.sync_copy(uid_hbm.at[pl.ds(base, BPW)], idx_v)
    pltpu.async_copy(comb.at[idx_v], rows_v, sem).wait()
    pltpu.sync_copy(rows_v, u_out.at[pl.ds(base, BPW)])
    pltpu.sync_copy(iid_hbm.at[pl.ds(base, BPW)], idx_v)
    pltpu.async_copy(comb.at[idx_v], rows_v, sem).wait()
    pltpu.sync_copy(rows_v, i_out.at[pl.ds(base, BPW)])


def _leaky(x):
    return jnp.where(x >= 0, x, 0.01 * x)


def _mlp_body(u_ref, i_ref, w0u_ref, w0i_ref, b0_ref,
              w1_ref, b1_ref, w2_ref, b2_ref, wg_ref, wh_ref, out_ref):
    p_mf = u_ref[:, 0:D]
    q_mf = i_ref[:, D:2 * D]
    p_mlp = u_ref[:, 2 * D:3 * D]
    q_mlp = i_ref[:, 3 * D:4 * D]
    h = jnp.dot(p_mlp, w0u_ref[...], preferred_element_type=jnp.float32)
    h = h + jnp.dot(q_mlp, w0i_ref[...], preferred_element_type=jnp.float32)
    h = _leaky(h + b0_ref[...])
    h = _leaky(jnp.dot(h, w1_ref[...], preferred_element_type=jnp.float32) + b1_ref[...])
    h = _leaky(jnp.dot(h, w2_ref[...], preferred_element_type=jnp.float32) + b2_ref[...])
    gmf = p_mf * q_mf
    out = jnp.dot(gmf, wg_ref[...], preferred_element_type=jnp.float32)
    out = out + jnp.dot(h, wh_ref[...], preferred_element_type=jnp.float32)
    out_ref[...] = out


_BLK = 2048


def _mlp_call(u_rows, i_rows, w0u, w0i, b0, w1t, b1, w2t, b2, wg, wh):
    grid = (BATCH // _BLK,)
    full = lambda shape: pl.BlockSpec(shape, lambda i: (0, 0))
    return pl.pallas_call(
        _mlp_body,
        grid=grid,
        in_specs=[
            pl.BlockSpec((_BLK, 4 * D), lambda i: (i, 0)),
            pl.BlockSpec((_BLK, 4 * D), lambda i: (i, 0)),
            full((D, 128)),
            full((D, 128)),
            full((1, 128)),
            full((128, 64)),
            full((1, 64)),
            full((64, 32)),
            full((1, 32)),
            full((D, 1)),
            full((32, 1)),
        ],
        out_specs=pl.BlockSpec((_BLK, 1), lambda i: (i, 0)),
        out_shape=jax.ShapeDtypeStruct((BATCH, 1), jnp.float32),
    )(u_rows, i_rows, w0u, w0i, b0, w1t, b1, w2t, b2, wg, wh)


def kernel(user_id, item_id, gmf_user, gmf_item, mlp_user, mlp_item,
           W0, b0, W1, b1, W2, b2, Wout):
    comb = _pack_call(gmf_user.T, gmf_item.T, mlp_user.T, mlp_item.T)
    u_rows, i_rows = _sc_gather(comb,
                                user_id.astype(jnp.int32),
                                item_id.astype(jnp.int32))
    return _mlp_call(u_rows, i_rows,
                     W0[:, :D].T, W0[:, D:].T, b0.reshape(1, -1),
                     W1.T, b1.reshape(1, -1), W2.T, b2.reshape(1, -1),
                     Wout[:, :D].T, Wout[:, D:].T)
